# baseline (speedup 1.0000x reference)
import functools

import jax
import jax.numpy as jnp
from jax import lax
from jax.experimental import pallas as pl
from jax.experimental.pallas import tpu as pltpu
from jax.experimental.pallas import tpu_sc as plsc

N = 10000
E = 320000
DIN = 128
H1 = 8
C1 = 8
COUT = 40

NC = 2
NS = 16
LANES = 16
NW = NC * NS
EPW = E // NW
CHUNK = 80
NCHUNK = EPW // CHUNK
GROUPS = CHUNK // LANES
RPT = 624
TAIL = N - NS * RPT

F1 = 80
F2 = 48
FD = 16


def _prep1_body(x_ref, w_ref, as_ref, ad_ref, src_out, dst_out):
    h = jnp.dot(x_ref[...], w_ref[...], preferred_element_type=jnp.float32)
    a_s = jnp.dot(h, as_ref[...], preferred_element_type=jnp.float32)
    a_d = jnp.dot(h, ad_ref[...], preferred_element_type=jnp.float32)
    amax = jnp.max(a_s, axis=0, keepdims=True)
    t = amax + a_d
    mt = jnp.maximum(t, 0.2 * t)
    z8 = jnp.zeros((N, 8), jnp.float32)
    src_out[...] = jnp.concatenate([h, a_s, z8], axis=1)
    dst_out[...] = jnp.concatenate([a_d, mt], axis=1)


def _mid_body(acc_ref, b1_ref, w2_ref, as2_ref, ad2_ref, rep_ref,
              src_out, dst_out):
    acc = acc_ref[0] + acc_ref[1]
    msg = acc[:, 0:64]
    den = acc[:, 64:72]
    den_rep = jnp.dot(den, rep_ref[...], preferred_element_type=jnp.float32)
    h1 = msg / (den_rep + 1e-16) + b1_ref[...]
    h1 = jnp.where(h1 > 0, h1, jnp.exp(jnp.minimum(h1, 0.0)) - 1.0)
    h2 = jnp.dot(h1, w2_ref[...], preferred_element_type=jnp.float32)
    a_s = jnp.dot(h2, as2_ref[...], preferred_element_type=jnp.float32)
    a_d = jnp.dot(h2, ad2_ref[...], preferred_element_type=jnp.float32)
    amax = jnp.max(a_s, axis=0, keepdims=True)
    t = amax + a_d
    mt = jnp.maximum(t, 0.2 * t)
    z7 = jnp.zeros((N, 7), jnp.float32)
    z14 = jnp.zeros((N, 14), jnp.float32)
    src_out[...] = jnp.concatenate([h2, a_s, z7], axis=1)
    dst_out[...] = jnp.concatenate([a_d, mt, z14], axis=1)


def _final_body(acc_ref, b2_ref, out_ref):
    acc = acc_ref[0] + acc_ref[1]
    msg = acc[:, 0:COUT]
    den = acc[:, COUT:COUT + 1]
    out_ref[...] = jax.nn.sigmoid(msg / (den + 1e-16) + b2_ref[...])


def _edge_kernel(F, H, C):
    mesh = plsc.VectorSubcoreMesh(
        core_axis_name="c", subcore_axis_name="s",
        num_cores=NC, num_subcores=NS)

    @functools.partial(
        pl.kernel,
        out_type=jax.ShapeDtypeStruct((NC * N, F), jnp.float32),
        mesh=mesh,
        compiler_params=pltpu.CompilerParams(
            use_tc_tiling_on_sc=False, needs_layout_passes=False),
        scratch_types=[
            pltpu.VMEM((NCHUNK, CHUNK), jnp.int32),
            pltpu.VMEM((NCHUNK, CHUNK), jnp.int32),
            pltpu.VMEM((CHUNK, F), jnp.float32),
            pltpu.VMEM((CHUNK, F), jnp.float32),
            pltpu.VMEM((CHUNK, FD), jnp.float32),
            pltpu.VMEM((CHUNK, FD), jnp.float32),
            pltpu.VMEM_SHARED((N, F), jnp.float32),
            pltpu.SemaphoreType.DMA,
            pltpu.SemaphoreType.DMA,
        ],
    )
    def k(srcf_hbm, dstf_hbm, srci_hbm, dsti_hbm, zero_hbm, out_hbm,
          srci_v, dsti_v, rows0, rows1, drows0, drows1, acc, gsem0, gsem1):
        cid = lax.axis_index("c")
        sid = lax.axis_index("s")
        wid = sid * NC + cid
        rows_b = (rows0, rows1)
        drows_b = (drows0, drows1)
        gsem_b = (gsem0, gsem1)

        pltpu.sync_copy(zero_hbm.at[pl.ds(sid * RPT, RPT)],
                        acc.at[pl.ds(sid * RPT, RPT)])

        @pl.when(sid == NS - 1)
        def _zero_tail():
            pltpu.sync_copy(zero_hbm.at[pl.ds(NS * RPT, TAIL)],
                            acc.at[pl.ds(NS * RPT, TAIL)])
        pltpu.sync_copy(srci_hbm.at[pl.ds(wid * NCHUNK, NCHUNK)], srci_v)
        pltpu.sync_copy(dsti_hbm.at[pl.ds(wid * NCHUNK, NCHUNK)], dsti_v)
        plsc.subcore_barrier()

        def start_g(j, b):
            pltpu.make_async_copy(
                srcf_hbm.at[srci_v.at[j]], rows_b[b], gsem_b[b]).start()
            pltpu.make_async_copy(
                dstf_hbm.at[dsti_v.at[j]], drows_b[b], gsem_b[b]).start()

        def wait_g(j, b):
            pltpu.make_async_copy(
                srcf_hbm.at[srci_v.at[j]], rows_b[b], gsem_b[b]).wait()
            pltpu.make_async_copy(
                dstf_hbm.at[dsti_v.at[j]], drows_b[b], gsem_b[b]).wait()

        def compute(b):
            rows = rows_b[b]
            drows = drows_b[b]
            for g in range(GROUPS):
                rowv = jnp.arange(g * LANES, (g + 1) * LANES, dtype=jnp.int32)
                exvs = []
                for h in range(H):
                    colh = jnp.full((LANES,), H * C + h, jnp.int32)
                    asv = plsc.load_gather(rows, [rowv, colh])
                    adv = plsc.load_gather(
                        drows, [rowv, jnp.full((LANES,), h, jnp.int32)])
                    mtv = plsc.load_gather(
                        drows, [rowv, jnp.full((LANES,), H + h, jnp.int32)])
                    t = asv + adv
                    e = jnp.maximum(t, 0.2 * t)
                    exv = jnp.exp(e - mtv)
                    plsc.store_scatter(rows, [rowv, colh], exv)
                    exvs.append(exv)
                for h in range(H):
                    for c in range(C):
                        colf = jnp.full((LANES,), h * C + c, jnp.int32)
                        hv = plsc.load_gather(rows, [rowv, colf])
                        plsc.store_scatter(rows, [rowv, colf], hv * exvs[h])

        def scatter(j, b):
            pltpu.sync_copy(rows_b[b], acc.at[dsti_v.at[j]], add=True)

        start_g(0, 0)

        def pair_body(p, carry):
            j0 = 2 * p
            start_g(j0 + 1, 1)
            wait_g(j0, 0)
            compute(0)
            scatter(j0, 0)
            start_g(j0 + 2, 0)
            wait_g(j0 + 1, 1)
            compute(1)
            scatter(j0 + 1, 1)
            return carry

        lax.fori_loop(0, (NCHUNK - 1) // 2, pair_body, 0)
        wait_g(NCHUNK - 1, 0)
        compute(0)
        scatter(NCHUNK - 1, 0)
        plsc.subcore_barrier()
        pltpu.sync_copy(acc.at[pl.ds(sid * RPT, RPT)],
                        out_hbm.at[pl.ds(cid * N + sid * RPT, RPT)])

        @pl.when(sid == NS - 1)
        def _write_tail():
            pltpu.sync_copy(acc.at[pl.ds(NS * RPT, TAIL)],
                            out_hbm.at[pl.ds(cid * N + NS * RPT, TAIL)])

    return k


_edge1 = _edge_kernel(F1, H1, C1)
_edge2 = _edge_kernel(F2, 1, COUT)


def _tc_call(body, out_shapes, *args):
    return pl.pallas_call(
        body,
        out_shape=out_shapes,
    )(*args)


def kernel(x, edge_index, W1, a_src1, a_dst1, b1, W2, a_src2, a_dst2, b2):
    src = edge_index[0].reshape(NW * NCHUNK, CHUNK)
    dst = edge_index[1].reshape(NW * NCHUNK, CHUNK)

    eye_h = (jnp.arange(H1 * C1)[:, None] // C1
             == jnp.arange(H1)[None, :]).astype(jnp.float32)
    As1 = a_src1.reshape(H1 * C1)[:, None] * eye_h
    Ad1 = a_dst1.reshape(H1 * C1)[:, None] * eye_h
    rep = eye_h.T

    srcf1, dstf1 = _tc_call(
        _prep1_body,
        [jax.ShapeDtypeStruct((N, F1), jnp.float32),
         jax.ShapeDtypeStruct((N, FD), jnp.float32)],
        x, W1, As1, Ad1)

    zero1 = jnp.zeros((N, F1), jnp.float32)
    acc1 = _edge1(srcf1, dstf1, src, dst, zero1).reshape(NC, N, F1)

    srcf2, dstf2 = _tc_call(
        _mid_body,
        [jax.ShapeDtypeStruct((N, F2), jnp.float32),
         jax.ShapeDtypeStruct((N, FD), jnp.float32)],
        acc1, b1.reshape(1, H1 * C1), W2, a_src2.T, a_dst2.T, rep)

    zero2 = jnp.zeros((N, F2), jnp.float32)
    acc2 = _edge2(srcf2, dstf2, src, dst, zero2).reshape(NC, N, F2)

    out = _tc_call(
        _final_body,
        jax.ShapeDtypeStruct((N, COUT), jnp.float32),
        acc2, b2.reshape(1, COUT))
    return out


# speedup vs baseline: 2.7701x; 2.7701x over previous
import functools

import jax
import jax.numpy as jnp
from jax import lax
from jax.experimental import pallas as pl
from jax.experimental.pallas import tpu as pltpu
from jax.experimental.pallas import tpu_sc as plsc

N = 10000
E = 320000
DIN = 128
H1 = 8
C1 = 8
COUT = 40

NC = 2
NS = 16
LANES = 16
NW = NC * NS
EPW = E // NW
CHUNK = 80
NCHUNK = EPW // CHUNK
GROUPS = CHUNK // LANES
RPT = 624
TAIL = N - NS * RPT

FA1 = 72
FA2 = 48
FS1 = 40
FS2 = 24
FD = 16


def _prep1_body(x_ref, w_ref, as_ref, ad_ref, src_out, dst_out):
    h = jnp.dot(x_ref[...], w_ref[...], preferred_element_type=jnp.float32)
    a_s = jnp.dot(h, as_ref[...], preferred_element_type=jnp.float32)
    a_d = jnp.dot(h, ad_ref[...], preferred_element_type=jnp.float32)
    amax = jnp.max(a_s, axis=0, keepdims=True)
    t = amax + a_d
    mt = jnp.maximum(t, 0.2 * t)
    src_out[...] = jnp.concatenate([h, a_s], axis=1)
    dst_out[...] = jnp.concatenate([a_d, mt], axis=1)


def _mid_body(acc_ref, b1_ref, w2_ref, as2_ref, ad2_ref, rep_ref,
              src_out, dst_out):
    acc = acc_ref[0] + acc_ref[1]
    msg = acc[:, 0:64]
    den = acc[:, 64:72]
    den_rep = jnp.dot(den, rep_ref[...], preferred_element_type=jnp.float32)
    h1 = msg / (den_rep + 1e-16) + b1_ref[...]
    h1 = jnp.where(h1 > 0, h1, jnp.exp(jnp.minimum(h1, 0.0)) - 1.0)
    h2 = jnp.dot(h1, w2_ref[...], preferred_element_type=jnp.float32)
    a_s = jnp.dot(h2, as2_ref[...], preferred_element_type=jnp.float32)
    a_d = jnp.dot(h2, ad2_ref[...], preferred_element_type=jnp.float32)
    amax = jnp.max(a_s, axis=0, keepdims=True)
    t = amax + a_d
    mt = jnp.maximum(t, 0.2 * t)
    z14 = jnp.zeros((N, 14), jnp.float32)
    src_out[...] = jnp.concatenate([h2, a_s], axis=1)
    dst_out[...] = jnp.concatenate([a_d, mt, z14], axis=1)


def _final_body(acc_ref, b2_ref, out_ref):
    acc = acc_ref[0] + acc_ref[1]
    msg = acc[:, 0:COUT]
    den = acc[:, COUT:COUT + 1]
    out_ref[...] = jax.nn.sigmoid(msg / (den + 1e-16) + b2_ref[...])


def _edge_kernel(FS, FA, H, C):
    NPAIR = (H * C) // 2
    mesh = plsc.VectorSubcoreMesh(
        core_axis_name="c", subcore_axis_name="s",
        num_cores=NC, num_subcores=NS)

    @functools.partial(
        pl.kernel,
        out_type=jax.ShapeDtypeStruct((NC * N, FA), jnp.float32),
        mesh=mesh,
        compiler_params=pltpu.CompilerParams(
            use_tc_tiling_on_sc=False, needs_layout_passes=False),
        scratch_types=[
            pltpu.VMEM((NCHUNK, CHUNK), jnp.int32),
            pltpu.VMEM((NCHUNK, CHUNK), jnp.int32),
            pltpu.VMEM((CHUNK, FS), jnp.int32),
            pltpu.VMEM((CHUNK, FS), jnp.int32),
            pltpu.VMEM((CHUNK, FS), jnp.int32),
            pltpu.VMEM((CHUNK, FD), jnp.float32),
            pltpu.VMEM((CHUNK, FD), jnp.float32),
            pltpu.VMEM((CHUNK, FD), jnp.float32),
            pltpu.VMEM((CHUNK, FA), jnp.float32),
            pltpu.VMEM((CHUNK, FA), jnp.float32),
            pltpu.VMEM((CHUNK, FA), jnp.float32),
            pltpu.VMEM_SHARED((N, FA), jnp.float32),
            pltpu.SemaphoreType.DMA,
            pltpu.SemaphoreType.DMA,
            pltpu.SemaphoreType.DMA,
            pltpu.SemaphoreType.DMA,
            pltpu.SemaphoreType.DMA,
            pltpu.SemaphoreType.DMA,
        ],
    )
    def k(srcf_hbm, dstf_hbm, srci_hbm, dsti_hbm, zero_hbm, out_hbm,
          srci_v, dsti_v, rows0, rows1, rows2, drows0, drows1, drows2,
          msg0, msg1, msg2, acc,
          gsem0, gsem1, gsem2, ssem0, ssem1, ssem2):
        cid = lax.axis_index("c")
        sid = lax.axis_index("s")
        wid = sid * NC + cid
        rows_b = (rows0, rows1, rows2)
        drows_b = (drows0, drows1, drows2)
        msg_b = (msg0, msg1, msg2)
        gsem_b = (gsem0, gsem1, gsem2)
        ssem_b = (ssem0, ssem1, ssem2)

        pltpu.sync_copy(zero_hbm.at[pl.ds(sid * RPT, RPT)],
                        acc.at[pl.ds(sid * RPT, RPT)])

        @pl.when(sid == NS - 1)
        def _zero_tail():
            pltpu.sync_copy(zero_hbm.at[pl.ds(NS * RPT, TAIL)],
                            acc.at[pl.ds(NS * RPT, TAIL)])
        pltpu.sync_copy(srci_hbm.at[pl.ds(wid * NCHUNK, NCHUNK)], srci_v)
        pltpu.sync_copy(dsti_hbm.at[pl.ds(wid * NCHUNK, NCHUNK)], dsti_v)
        plsc.subcore_barrier()

        def start_g(j, b):
            pltpu.make_async_copy(
                srcf_hbm.at[srci_v.at[j]], rows_b[b], gsem_b[b]).start()
            pltpu.make_async_copy(
                dstf_hbm.at[dsti_v.at[j]], drows_b[b], gsem_b[b]).start()

        def wait_g(j, b):
            pltpu.make_async_copy(
                srcf_hbm.at[srci_v.at[j]], rows_b[b], gsem_b[b]).wait()
            pltpu.make_async_copy(
                dstf_hbm.at[dsti_v.at[j]], drows_b[b], gsem_b[b]).wait()

        PBLK = NPAIR // H if H > 1 else 5
        NCB = NPAIR // PBLK if H == 1 else 1
        DIVB = H * NCB
        DIVB_BITS = DIVB.bit_length() - 1
        assert 1 << DIVB_BITS == DIVB and (H == 1 or NCB == 1)

        zlane = jnp.zeros((LANES,), jnp.float32)
        for b in range(3):
            for g in range(GROUPS):
                rowz = jnp.arange(g * LANES, (g + 1) * LANES,
                                  dtype=jnp.int32)
                for col in range(H * C + H, FA):
                    plsc.store_scatter(
                        msg_b[b], [rowz, jnp.broadcast_to(col, (LANES,))],
                        zlane)

        def compute(b):
            rows = rows_b[b]
            drows = drows_b[b]
            msg = msg_b[b]
            iota = lax.iota(jnp.int32, LANES)

            @plsc.parallel_loop(0, GROUPS * H, unroll=min(8, GROUPS * H))
            def _phase_a(i):
                if H > 1:
                    g = lax.shift_right_logical(i, DIVB_BITS)
                    h = lax.bitwise_and(i, H - 1)
                else:
                    g, h = i, 0
                rowv = g * LANES + iota
                asv = plsc.bitcast(
                    plsc.load_gather(
                        rows, [rowv, jnp.broadcast_to(NPAIR + h, (LANES,))]),
                    jnp.float32)
                adv = plsc.load_gather(
                    drows, [rowv, jnp.broadcast_to(h, (LANES,))])
                mtv = plsc.load_gather(
                    drows, [rowv, jnp.broadcast_to(H + h, (LANES,))])
                t = asv + adv
                e = jnp.maximum(t, 0.2 * t)
                exv = jnp.exp(e - mtv)
                plsc.store_scatter(
                    msg, [rowv, jnp.broadcast_to(H * C + h, (LANES,))], exv)

            @plsc.parallel_loop(0, GROUPS * DIVB, unroll=8)
            def _phase_b(i):
                g = lax.shift_right_logical(i, DIVB_BITS)
                r = lax.bitwise_and(i, DIVB - 1)
                h = r if NCB == 1 else 0
                base = h * PBLK if NCB == 1 else r * PBLK
                rowv = g * LANES + iota
                exv = plsc.load_gather(
                    msg, [rowv, jnp.broadcast_to(H * C + h, (LANES,))])
                for p in range(PBLK):
                    w = plsc.load_gather(
                        rows, [rowv, jnp.broadcast_to(base + p, (LANES,))])
                    lo = plsc.bitcast(w << 16, jnp.float32)
                    hi = plsc.bitcast(w & jnp.int32(-65536), jnp.float32)
                    fc = 2 * (base + p)
                    plsc.store_scatter(
                        msg, [rowv, jnp.broadcast_to(fc, (LANES,))],
                        lo * exv)
                    plsc.store_scatter(
                        msg, [rowv, jnp.broadcast_to(fc + 1, (LANES,))],
                        hi * exv)

        def start_s(j, b):
            pltpu.make_async_copy(
                msg_b[b], acc.at[dsti_v.at[j]], ssem_b[b]).start(add=True)

        def wait_s(j, b):
            pltpu.make_async_copy(
                msg_b[b], acc.at[dsti_v.at[j]], ssem_b[b]).wait()

        def slot(t, b):
            j = 3 * t + b
            wait_g(j, b)
            compute(b)
            start_s(j, b)
            bp = (b + 2) % 3

            @pl.when(j >= 1)
            def _drain_prev():
                wait_s(j - 1, bp)

            @pl.when(j + 2 < NCHUNK)
            def _prefetch():
                start_g(j + 2, bp)

        start_g(0, 0)
        start_g(1, 1)

        def triple_body(t, carry):
            slot(t, 0)
            slot(t, 1)
            slot(t, 2)
            return carry

        NTRIPLE = NCHUNK // 3
        lax.fori_loop(0, NTRIPLE, triple_body, 0)
        slot(NTRIPLE, 0)
        slot(NTRIPLE, 1)
        wait_s(NCHUNK - 1, (NCHUNK - 1) % 3)
        plsc.subcore_barrier()
        pltpu.sync_copy(acc.at[pl.ds(sid * RPT, RPT)],
                        out_hbm.at[pl.ds(cid * N + sid * RPT, RPT)])

        @pl.when(sid == NS - 1)
        def _write_tail():
            pltpu.sync_copy(acc.at[pl.ds(NS * RPT, TAIL)],
                            out_hbm.at[pl.ds(cid * N + NS * RPT, TAIL)])

    return k


_edge1 = _edge_kernel(FS1, FA1, H1, C1)
_edge2 = _edge_kernel(FS2, FA2, 1, COUT)


def _pack_src(srcf, nfeat):
    hb = srcf[:, :nfeat].astype(jnp.bfloat16)
    pairs = lax.bitcast_convert_type(
        hb.reshape(N, nfeat // 2, 2), jnp.int32)
    asb = lax.bitcast_convert_type(srcf[:, nfeat:], jnp.int32)
    return jnp.concatenate([pairs, asb], axis=1)


def _tc_call(body, out_shapes, *args):
    return pl.pallas_call(
        body,
        out_shape=out_shapes,
    )(*args)


def kernel(x, edge_index, W1, a_src1, a_dst1, b1, W2, a_src2, a_dst2, b2):
    src = edge_index[0].reshape(NW * NCHUNK, CHUNK)
    dst = edge_index[1].reshape(NW * NCHUNK, CHUNK)

    eye_h = (jnp.arange(H1 * C1)[:, None] // C1
             == jnp.arange(H1)[None, :]).astype(jnp.float32)
    As1 = a_src1.reshape(H1 * C1)[:, None] * eye_h
    Ad1 = a_dst1.reshape(H1 * C1)[:, None] * eye_h
    rep = eye_h.T

    srcf1, dstf1 = _tc_call(
        _prep1_body,
        [jax.ShapeDtypeStruct((N, H1 * C1 + H1), jnp.float32),
         jax.ShapeDtypeStruct((N, FD), jnp.float32)],
        x, W1, As1, Ad1)
    srci1 = _pack_src(srcf1, H1 * C1)

    zero1 = jnp.zeros((N, FA1), jnp.float32)
    acc1 = _edge1(srci1, dstf1, src, dst, zero1).reshape(NC, N, FA1)

    srcf2, dstf2 = _tc_call(
        _mid_body,
        [jax.ShapeDtypeStruct((N, COUT + 1), jnp.float32),
         jax.ShapeDtypeStruct((N, FD), jnp.float32)],
        acc1, b1.reshape(1, H1 * C1), W2, a_src2.T, a_dst2.T, rep)
    srci2 = jnp.concatenate(
        [_pack_src(srcf2, COUT), jnp.zeros((N, 3), jnp.int32)], axis=1)

    zero2 = jnp.zeros((N, FA2), jnp.float32)
    acc2 = _edge2(srci2, dstf2, src, dst, zero2).reshape(NC, N, FA2)

    out = _tc_call(
        _final_body,
        jax.ShapeDtypeStruct((N, COUT), jnp.float32),
        acc2, b2.reshape(1, COUT))
    return out


# speedup vs baseline: 3.0478x; 1.1002x over previous
import functools

import jax
import jax.numpy as jnp
from jax import lax
from jax.experimental import pallas as pl
from jax.experimental.pallas import tpu as pltpu
from jax.experimental.pallas import tpu_sc as plsc

N = 10000
E = 320000
DIN = 128
H1 = 8
C1 = 8
COUT = 40

NC = 2
NS = 16
LANES = 16
NW = NC * NS
EPW = E // NW
CHUNK = 80
NCHUNK = EPW // CHUNK
GROUPS = CHUNK // LANES
RPT = 624
TAIL = N - NS * RPT

FA1 = 72
FA2 = 48
FS1 = 40
FS2 = 24
FD = 16


def _pack_halves(v):
    k = v.shape[1] // 2
    b = lax.bitcast_convert_type(v, jnp.int32)
    r = b + 0x7FFF + (lax.shift_right_logical(b, 16) & 1)
    lo = lax.shift_right_logical(r[:, :k], 16)
    hi = r[:, k:] & jnp.int32(-65536)
    return lo | hi


def _prep1_body(x_ref, w_ref, as_ref, ad_ref, src_out, dst_out):
    h = jnp.dot(x_ref[...], w_ref[...], preferred_element_type=jnp.float32)
    a_s = jnp.dot(h, as_ref[...], preferred_element_type=jnp.float32)
    a_d = jnp.dot(h, ad_ref[...], preferred_element_type=jnp.float32)
    amax = jnp.max(a_s, axis=0, keepdims=True)
    t = amax + a_d
    mt = jnp.maximum(t, 0.2 * t)
    src_out[...] = jnp.concatenate(
        [_pack_halves(h), lax.bitcast_convert_type(a_s, jnp.int32)], axis=1)
    dst_out[...] = jnp.concatenate([a_d, mt], axis=1)


def _mid_body(acc_ref, b1_ref, w2_ref, as2_ref, ad2_ref, rep_ref,
              src_out, dst_out):
    acc = acc_ref[0] + acc_ref[1]
    msg = acc[:, 0:64]
    den = acc[:, 64:72]
    den_rep = jnp.dot(den, rep_ref[...], preferred_element_type=jnp.float32)
    h1 = msg / (den_rep + 1e-16) + b1_ref[...]
    h1 = jnp.where(h1 > 0, h1, jnp.exp(jnp.minimum(h1, 0.0)) - 1.0)
    h2 = jnp.dot(h1, w2_ref[...], preferred_element_type=jnp.float32)
    a_s = jnp.dot(h2, as2_ref[...], preferred_element_type=jnp.float32)
    a_d = jnp.dot(h2, ad2_ref[...], preferred_element_type=jnp.float32)
    amax = jnp.max(a_s, axis=0, keepdims=True)
    t = amax + a_d
    mt = jnp.maximum(t, 0.2 * t)
    z14 = jnp.zeros((N, 14), jnp.float32)
    z3 = jnp.zeros((N, 3), jnp.int32)
    src_out[...] = jnp.concatenate(
        [_pack_halves(h2), lax.bitcast_convert_type(a_s, jnp.int32), z3],
        axis=1)
    dst_out[...] = jnp.concatenate([a_d, mt, z14], axis=1)


def _final_body(acc_ref, b2_ref, out_ref):
    acc = acc_ref[0] + acc_ref[1]
    msg = acc[:, 0:COUT]
    den = acc[:, COUT:COUT + 1]
    out_ref[...] = jax.nn.sigmoid(msg / (den + 1e-16) + b2_ref[...])


def _edge_kernel(FS, FA, H, C):
    NPAIR = (H * C) // 2
    mesh = plsc.VectorSubcoreMesh(
        core_axis_name="c", subcore_axis_name="s",
        num_cores=NC, num_subcores=NS)

    @functools.partial(
        pl.kernel,
        out_type=jax.ShapeDtypeStruct((NC * N, FA), jnp.float32),
        mesh=mesh,
        compiler_params=pltpu.CompilerParams(
            use_tc_tiling_on_sc=False, needs_layout_passes=False),
        scratch_types=[
            pltpu.VMEM((NCHUNK, CHUNK), jnp.int32),
            pltpu.VMEM((NCHUNK, CHUNK), jnp.int32),
            pltpu.VMEM((CHUNK, FS), jnp.int32),
            pltpu.VMEM((CHUNK, FS), jnp.int32),
            pltpu.VMEM((CHUNK, FS), jnp.int32),
            pltpu.VMEM((CHUNK, FD), jnp.float32),
            pltpu.VMEM((CHUNK, FD), jnp.float32),
            pltpu.VMEM((CHUNK, FD), jnp.float32),
            pltpu.VMEM((CHUNK, FA), jnp.float32),
            pltpu.VMEM((CHUNK, FA), jnp.float32),
            pltpu.VMEM((CHUNK, FA), jnp.float32),
            pltpu.VMEM_SHARED((N, FA), jnp.float32),
            pltpu.SemaphoreType.DMA,
            pltpu.SemaphoreType.DMA,
            pltpu.SemaphoreType.DMA,
            pltpu.SemaphoreType.DMA,
            pltpu.SemaphoreType.DMA,
            pltpu.SemaphoreType.DMA,
        ],
    )
    def k(srcf_hbm, dstf_hbm, srci_hbm, dsti_hbm, zero_hbm, out_hbm,
          srci_v, dsti_v, rows0, rows1, rows2, drows0, drows1, drows2,
          msg0, msg1, msg2, acc,
          gsem0, gsem1, gsem2, ssem0, ssem1, ssem2):
        cid = lax.axis_index("c")
        sid = lax.axis_index("s")
        wid = sid * NC + cid
        rows_b = (rows0, rows1, rows2)
        drows_b = (drows0, drows1, drows2)
        msg_b = (msg0, msg1, msg2)
        gsem_b = (gsem0, gsem1, gsem2)
        ssem_b = (ssem0, ssem1, ssem2)

        pltpu.sync_copy(zero_hbm.at[pl.ds(sid * RPT, RPT)],
                        acc.at[pl.ds(sid * RPT, RPT)])

        @pl.when(sid == NS - 1)
        def _zero_tail():
            pltpu.sync_copy(zero_hbm.at[pl.ds(NS * RPT, TAIL)],
                            acc.at[pl.ds(NS * RPT, TAIL)])
        pltpu.sync_copy(srci_hbm.at[pl.ds(wid * NCHUNK, NCHUNK)], srci_v)
        pltpu.sync_copy(dsti_hbm.at[pl.ds(wid * NCHUNK, NCHUNK)], dsti_v)
        plsc.subcore_barrier()

        def start_g(j, b):
            pltpu.make_async_copy(
                srcf_hbm.at[srci_v.at[j]], rows_b[b], gsem_b[b]).start()
            pltpu.make_async_copy(
                dstf_hbm.at[dsti_v.at[j]], drows_b[b], gsem_b[b]).start()

        def wait_g(j, b):
            pltpu.make_async_copy(
                srcf_hbm.at[srci_v.at[j]], rows_b[b], gsem_b[b]).wait()
            pltpu.make_async_copy(
                dstf_hbm.at[dsti_v.at[j]], drows_b[b], gsem_b[b]).wait()

        HBITS = H.bit_length() - 1
        PBLK = NPAIR // 4
        assert 1 << HBITS == H and NPAIR % 4 == 0

        zlane = jnp.zeros((LANES,), jnp.float32)
        for b in range(3):
            for g in range(GROUPS):
                rowz = jnp.arange(g * LANES, (g + 1) * LANES,
                                  dtype=jnp.int32)
                for col in range(H * C + H, FA):
                    plsc.store_scatter(
                        msg_b[b], [rowz, jnp.broadcast_to(col, (LANES,))],
                        zlane)

        def compute(b):
            rows = rows_b[b]
            drows = drows_b[b]
            msg = msg_b[b]
            iota = lax.iota(jnp.int32, LANES)

            @plsc.parallel_loop(0, GROUPS * H, unroll=min(8, GROUPS * H))
            def _phase_a(i):
                if H > 1:
                    g = lax.shift_right_logical(i, HBITS)
                    h = lax.bitwise_and(i, H - 1)
                else:
                    g, h = i, 0
                rowv = g * LANES + iota
                asv = plsc.bitcast(
                    plsc.load_gather(
                        rows, [rowv, jnp.broadcast_to(NPAIR + h, (LANES,))]),
                    jnp.float32)
                adv = plsc.load_gather(
                    drows, [rowv, jnp.broadcast_to(h, (LANES,))])
                mtv = plsc.load_gather(
                    drows, [rowv, jnp.broadcast_to(H + h, (LANES,))])
                t = asv + adv
                e = jnp.maximum(t, 0.2 * t)
                exv = jnp.exp(e - mtv)
                plsc.store_scatter(
                    msg, [rowv, jnp.broadcast_to(H * C + h, (LANES,))], exv)

            @plsc.parallel_loop(0, GROUPS * 4, unroll=8)
            def _phase_b(i):
                g = lax.shift_right_logical(i, 2)
                r = lax.bitwise_and(i, 3)
                rowv = g * LANES + iota
                if H > 1:
                    ex_lo = plsc.load_gather(
                        msg, [rowv, jnp.broadcast_to(H * C + r, (LANES,))])
                    ex_hi = plsc.load_gather(
                        msg, [rowv,
                              jnp.broadcast_to(H * C + H // 2 + r,
                                               (LANES,))])
                else:
                    ex_lo = plsc.load_gather(
                        msg, [rowv, jnp.broadcast_to(H * C, (LANES,))])
                    ex_hi = ex_lo
                base = r * PBLK
                for p in range(PBLK):
                    w = plsc.load_gather(
                        rows, [rowv, jnp.broadcast_to(base + p, (LANES,))])
                    lo = plsc.bitcast(w << 16, jnp.float32)
                    hi = plsc.bitcast(w & jnp.int32(-65536), jnp.float32)
                    plsc.store_scatter(
                        msg, [rowv, jnp.broadcast_to(base + p, (LANES,))],
                        lo * ex_lo)
                    plsc.store_scatter(
                        msg, [rowv,
                              jnp.broadcast_to(NPAIR + base + p, (LANES,))],
                        hi * ex_hi)

        def start_s(j, b):
            pltpu.make_async_copy(
                msg_b[b], acc.at[dsti_v.at[j]], ssem_b[b]).start(add=True)

        def wait_s(j, b):
            pltpu.make_async_copy(
                msg_b[b], acc.at[dsti_v.at[j]], ssem_b[b]).wait()

        def slot(t, b):
            j = 3 * t + b
            wait_g(j, b)
            compute(b)
            start_s(j, b)
            bp = (b + 2) % 3

            @pl.when(j >= 1)
            def _drain_prev():
                wait_s(j - 1, bp)

            @pl.when(j + 2 < NCHUNK)
            def _prefetch():
                start_g(j + 2, bp)

        start_g(0, 0)
        start_g(1, 1)

        def triple_body(t, carry):
            slot(t, 0)
            slot(t, 1)
            slot(t, 2)
            return carry

        NTRIPLE = NCHUNK // 3
        lax.fori_loop(0, NTRIPLE, triple_body, 0)
        slot(NTRIPLE, 0)
        slot(NTRIPLE, 1)
        wait_s(NCHUNK - 1, (NCHUNK - 1) % 3)
        plsc.subcore_barrier()
        pltpu.sync_copy(acc.at[pl.ds(sid * RPT, RPT)],
                        out_hbm.at[pl.ds(cid * N + sid * RPT, RPT)])

        @pl.when(sid == NS - 1)
        def _write_tail():
            pltpu.sync_copy(acc.at[pl.ds(NS * RPT, TAIL)],
                            out_hbm.at[pl.ds(cid * N + NS * RPT, TAIL)])

    return k


_edge1 = _edge_kernel(FS1, FA1, H1, C1)
_edge2 = _edge_kernel(FS2, FA2, 1, COUT)


def _tc_call(body, out_shapes, *args):
    return pl.pallas_call(
        body,
        out_shape=out_shapes,
    )(*args)


def kernel(x, edge_index, W1, a_src1, a_dst1, b1, W2, a_src2, a_dst2, b2):
    src = edge_index[0].reshape(NW * NCHUNK, CHUNK)
    dst = edge_index[1].reshape(NW * NCHUNK, CHUNK)

    eye_h = (jnp.arange(H1 * C1)[:, None] // C1
             == jnp.arange(H1)[None, :]).astype(jnp.float32)
    As1 = a_src1.reshape(H1 * C1)[:, None] * eye_h
    Ad1 = a_dst1.reshape(H1 * C1)[:, None] * eye_h
    rep = eye_h.T

    srci1, dstf1 = _tc_call(
        _prep1_body,
        [jax.ShapeDtypeStruct((N, FS1), jnp.int32),
         jax.ShapeDtypeStruct((N, FD), jnp.float32)],
        x, W1, As1, Ad1)

    zero1 = jnp.zeros((N, FA1), jnp.float32)
    acc1 = _edge1(srci1, dstf1, src, dst, zero1).reshape(NC, N, FA1)

    srci2, dstf2 = _tc_call(
        _mid_body,
        [jax.ShapeDtypeStruct((N, FS2), jnp.int32),
         jax.ShapeDtypeStruct((N, FD), jnp.float32)],
        acc1, b1.reshape(1, H1 * C1), W2, a_src2.T, a_dst2.T, rep)

    zero2 = jnp.zeros((N, FA2), jnp.float32)
    acc2 = _edge2(srci2, dstf2, src, dst, zero2).reshape(NC, N, FA2)

    out = _tc_call(
        _final_body,
        jax.ShapeDtypeStruct((N, COUT), jnp.float32),
        acc2, b2.reshape(1, COUT))
    return out


# speedup vs baseline: 3.2179x; 1.0558x over previous
import functools

import jax
import jax.numpy as jnp
from jax import lax
from jax.experimental import pallas as pl
from jax.experimental.pallas import tpu as pltpu
from jax.experimental.pallas import tpu_sc as plsc

N = 10000
E = 320000
DIN = 128
H1 = 8
C1 = 8
COUT = 40

NC = 2
NS = 16
LANES = 16
NW = NC * NS
EPW = E // NW
CHUNK = 80
NCHUNK = EPW // CHUNK
GROUPS = CHUNK // LANES
RPT = 624
TAIL = N - NS * RPT

FA1 = 72
FA2 = 48
FS1 = 40
FS2 = 24
FD = 16


def _pack_halves(v):
    k = v.shape[1] // 2
    b = lax.bitcast_convert_type(v, jnp.int32)
    r = b + 0x7FFF + (lax.shift_right_logical(b, 16) & 1)
    lo = lax.shift_right_logical(r[:, :k], 16)
    hi = r[:, k:] & jnp.int32(-65536)
    return lo | hi


def _prep1_body(x_ref, w_ref, as_ref, ad_ref, src_out, dst_out):
    h = jnp.dot(x_ref[...], w_ref[...], preferred_element_type=jnp.float32)
    a_s = jnp.dot(h, as_ref[...], preferred_element_type=jnp.float32)
    a_d = jnp.dot(h, ad_ref[...], preferred_element_type=jnp.float32)
    amax = jnp.max(a_s, axis=0, keepdims=True)
    t = amax + a_d
    mt = jnp.maximum(t, 0.2 * t)
    src_out[...] = jnp.concatenate(
        [_pack_halves(h), lax.bitcast_convert_type(a_s, jnp.int32)], axis=1)
    dst_out[...] = jnp.concatenate([a_d, mt], axis=1)


def _mid_body(acc_ref, b1_ref, w2_ref, as2_ref, ad2_ref, rep_ref,
              src_out, dst_out):
    acc = acc_ref[0] + acc_ref[1]
    msg = acc[:, 0:64]
    den = acc[:, 64:72]
    den_rep = jnp.dot(den, rep_ref[...], preferred_element_type=jnp.float32)
    h1 = msg / (den_rep + 1e-16) + b1_ref[...]
    h1 = jnp.where(h1 > 0, h1, jnp.exp(jnp.minimum(h1, 0.0)) - 1.0)
    h2 = jnp.dot(h1, w2_ref[...], preferred_element_type=jnp.float32)
    a_s = jnp.dot(h2, as2_ref[...], preferred_element_type=jnp.float32)
    a_d = jnp.dot(h2, ad2_ref[...], preferred_element_type=jnp.float32)
    amax = jnp.max(a_s, axis=0, keepdims=True)
    t = amax + a_d
    mt = jnp.maximum(t, 0.2 * t)
    z14 = jnp.zeros((N, 14), jnp.float32)
    z3 = jnp.zeros((N, 3), jnp.int32)
    src_out[...] = jnp.concatenate(
        [_pack_halves(h2), lax.bitcast_convert_type(a_s, jnp.int32), z3],
        axis=1)
    dst_out[...] = jnp.concatenate([a_d, mt, z14], axis=1)


def _final_body(acc_ref, b2_ref, out_ref):
    acc = acc_ref[0] + acc_ref[1]
    msg = acc[:, 0:COUT]
    den = acc[:, COUT:COUT + 1]
    out_ref[...] = jax.nn.sigmoid(msg / (den + 1e-16) + b2_ref[...])


def _edge_kernel(FS, FA, H, C):
    NPAIR = (H * C) // 2
    mesh = plsc.VectorSubcoreMesh(
        core_axis_name="c", subcore_axis_name="s",
        num_cores=NC, num_subcores=NS)

    @functools.partial(
        pl.kernel,
        out_type=jax.ShapeDtypeStruct((NC * N, FA), jnp.float32),
        mesh=mesh,
        compiler_params=pltpu.CompilerParams(
            use_tc_tiling_on_sc=False, needs_layout_passes=False),
        scratch_types=[
            pltpu.VMEM((NCHUNK, CHUNK), jnp.int32),
            pltpu.VMEM((NCHUNK, CHUNK), jnp.int32),
            pltpu.VMEM((CHUNK, FS), jnp.int32),
            pltpu.VMEM((CHUNK, FS), jnp.int32),
            pltpu.VMEM((CHUNK, FS), jnp.int32),
            pltpu.VMEM((CHUNK, FD), jnp.float32),
            pltpu.VMEM((CHUNK, FD), jnp.float32),
            pltpu.VMEM((CHUNK, FD), jnp.float32),
            pltpu.VMEM((CHUNK, FA), jnp.float32),
            pltpu.VMEM((CHUNK, FA), jnp.float32),
            pltpu.VMEM((CHUNK, FA), jnp.float32),
            pltpu.VMEM_SHARED((N, FA), jnp.float32),
            pltpu.SemaphoreType.DMA,
            pltpu.SemaphoreType.DMA,
            pltpu.SemaphoreType.DMA,
            pltpu.SemaphoreType.DMA,
            pltpu.SemaphoreType.DMA,
            pltpu.SemaphoreType.DMA,
        ],
    )
    def k(srcf_hbm, dstf_hbm, srci_hbm, dsti_hbm, zero_hbm, out_hbm,
          srci_v, dsti_v, rows0, rows1, rows2, drows0, drows1, drows2,
          msg0, msg1, msg2, acc,
          gsem0, gsem1, gsem2, ssem0, ssem1, ssem2):
        cid = lax.axis_index("c")
        sid = lax.axis_index("s")
        wid = sid * NC + cid
        rows_b = (rows0, rows1, rows2)
        drows_b = (drows0, drows1, drows2)
        msg_b = (msg0, msg1, msg2)
        gsem_b = (gsem0, gsem1, gsem2)
        ssem_b = (ssem0, ssem1, ssem2)

        pltpu.sync_copy(zero_hbm.at[pl.ds(sid * RPT, RPT)],
                        acc.at[pl.ds(sid * RPT, RPT)])

        @pl.when(sid == NS - 1)
        def _zero_tail():
            pltpu.sync_copy(zero_hbm.at[pl.ds(NS * RPT, TAIL)],
                            acc.at[pl.ds(NS * RPT, TAIL)])
        pltpu.sync_copy(srci_hbm.at[pl.ds(wid * NCHUNK, NCHUNK)], srci_v)
        pltpu.sync_copy(dsti_hbm.at[pl.ds(wid * NCHUNK, NCHUNK)], dsti_v)
        plsc.subcore_barrier()

        def start_g(j, b):
            pltpu.make_async_copy(
                srcf_hbm.at[srci_v.at[j]], rows_b[b], gsem_b[b]).start()
            pltpu.make_async_copy(
                dstf_hbm.at[dsti_v.at[j]], drows_b[b], gsem_b[b]).start()

        def wait_g(j, b):
            pltpu.make_async_copy(
                srcf_hbm.at[srci_v.at[j]], rows_b[b], gsem_b[b]).wait()
            pltpu.make_async_copy(
                dstf_hbm.at[dsti_v.at[j]], drows_b[b], gsem_b[b]).wait()

        HBITS = H.bit_length() - 1
        DIVB = 8 if H > 1 else 4
        BBITS = 3 if H > 1 else 2
        PBLK = NPAIR // DIVB
        assert 1 << HBITS == H and NPAIR % DIVB == 0

        zlane = jnp.zeros((LANES,), jnp.float32)
        for b in range(3):
            for g in range(GROUPS):
                rowz = jnp.arange(g * LANES, (g + 1) * LANES,
                                  dtype=jnp.int32)
                for col in range(H * C + H, FA):
                    plsc.store_scatter(
                        msg_b[b], [rowz, jnp.broadcast_to(col, (LANES,))],
                        zlane)

        def compute(b):
            rows = rows_b[b]
            drows = drows_b[b]
            msg = msg_b[b]
            iota = lax.iota(jnp.int32, LANES)

            @plsc.parallel_loop(0, GROUPS * H, unroll=min(8, GROUPS * H))
            def _phase_a(i):
                if H > 1:
                    g = lax.shift_right_logical(i, HBITS)
                    h = lax.bitwise_and(i, H - 1)
                else:
                    g, h = i, 0
                rowv = g * LANES + iota
                asv = plsc.bitcast(
                    plsc.load_gather(
                        rows, [rowv, jnp.broadcast_to(NPAIR + h, (LANES,))]),
                    jnp.float32)
                adv = plsc.load_gather(
                    drows, [rowv, jnp.broadcast_to(h, (LANES,))])
                mtv = plsc.load_gather(
                    drows, [rowv, jnp.broadcast_to(H + h, (LANES,))])
                t = asv + adv
                e = jnp.maximum(t, 0.2 * t)
                exv = jnp.exp(e - mtv)
                plsc.store_scatter(
                    msg, [rowv, jnp.broadcast_to(H * C + h, (LANES,))], exv)

            @plsc.parallel_loop(0, GROUPS * DIVB, unroll=8)
            def _phase_b(i):
                g = lax.shift_right_logical(i, BBITS)
                r = lax.bitwise_and(i, DIVB - 1)
                rowv = g * LANES + iota
                if H > 1:
                    hh = lax.shift_right_logical(r, 1)
                    ex_lo = plsc.load_gather(
                        msg, [rowv, jnp.broadcast_to(H * C + hh, (LANES,))])
                    ex_hi = plsc.load_gather(
                        msg, [rowv,
                              jnp.broadcast_to(H * C + H // 2 + hh,
                                               (LANES,))])
                else:
                    ex_lo = plsc.load_gather(
                        msg, [rowv, jnp.broadcast_to(H * C, (LANES,))])
                    ex_hi = ex_lo
                base = r * PBLK
                for p in range(PBLK):
                    w = plsc.load_gather(
                        rows, [rowv, jnp.broadcast_to(base + p, (LANES,))])
                    lo = plsc.bitcast(w << 16, jnp.float32)
                    hi = plsc.bitcast(w & jnp.int32(-65536), jnp.float32)
                    plsc.store_scatter(
                        msg, [rowv, jnp.broadcast_to(base + p, (LANES,))],
                        lo * ex_lo)
                    plsc.store_scatter(
                        msg, [rowv,
                              jnp.broadcast_to(NPAIR + base + p, (LANES,))],
                        hi * ex_hi)

        def start_s(j, b):
            pltpu.make_async_copy(
                msg_b[b], acc.at[dsti_v.at[j]], ssem_b[b]).start(add=True)

        def wait_s(j, b):
            pltpu.make_async_copy(
                msg_b[b], acc.at[dsti_v.at[j]], ssem_b[b]).wait()

        def slot(t, b):
            j = 3 * t + b
            wait_g(j, b)
            compute(b)
            start_s(j, b)
            bp = (b + 2) % 3

            @pl.when(j >= 1)
            def _drain_prev():
                wait_s(j - 1, bp)

            @pl.when(j + 2 < NCHUNK)
            def _prefetch():
                start_g(j + 2, bp)

        start_g(0, 0)
        start_g(1, 1)

        def triple_body(t, carry):
            slot(t, 0)
            slot(t, 1)
            slot(t, 2)
            return carry

        NTRIPLE = NCHUNK // 3
        lax.fori_loop(0, NTRIPLE, triple_body, 0)
        slot(NTRIPLE, 0)
        slot(NTRIPLE, 1)
        wait_s(NCHUNK - 1, (NCHUNK - 1) % 3)
        plsc.subcore_barrier()
        pltpu.sync_copy(acc.at[pl.ds(sid * RPT, RPT)],
                        out_hbm.at[pl.ds(cid * N + sid * RPT, RPT)])

        @pl.when(sid == NS - 1)
        def _write_tail():
            pltpu.sync_copy(acc.at[pl.ds(NS * RPT, TAIL)],
                            out_hbm.at[pl.ds(cid * N + NS * RPT, TAIL)])

    return k


_edge1 = _edge_kernel(FS1, FA1, H1, C1)
_edge2 = _edge_kernel(FS2, FA2, 1, COUT)


def _tc_call(body, out_shapes, *args):
    return pl.pallas_call(
        body,
        out_shape=out_shapes,
    )(*args)


def kernel(x, edge_index, W1, a_src1, a_dst1, b1, W2, a_src2, a_dst2, b2):
    src = edge_index[0].reshape(NW * NCHUNK, CHUNK)
    dst = edge_index[1].reshape(NW * NCHUNK, CHUNK)

    eye_h = (jnp.arange(H1 * C1)[:, None] // C1
             == jnp.arange(H1)[None, :]).astype(jnp.float32)
    As1 = a_src1.reshape(H1 * C1)[:, None] * eye_h
    Ad1 = a_dst1.reshape(H1 * C1)[:, None] * eye_h
    rep = eye_h.T

    srci1, dstf1 = _tc_call(
        _prep1_body,
        [jax.ShapeDtypeStruct((N, FS1), jnp.int32),
         jax.ShapeDtypeStruct((N, FD), jnp.float32)],
        x, W1, As1, Ad1)

    zero1 = jnp.zeros((N, FA1), jnp.float32)
    acc1 = _edge1(srci1, dstf1, src, dst, zero1).reshape(NC, N, FA1)

    srci2, dstf2 = _tc_call(
        _mid_body,
        [jax.ShapeDtypeStruct((N, FS2), jnp.int32),
         jax.ShapeDtypeStruct((N, FD), jnp.float32)],
        acc1, b1.reshape(1, H1 * C1), W2, a_src2.T, a_dst2.T, rep)

    zero2 = jnp.zeros((N, FA2), jnp.float32)
    acc2 = _edge2(srci2, dstf2, src, dst, zero2).reshape(NC, N, FA2)

    out = _tc_call(
        _final_body,
        jax.ShapeDtypeStruct((N, COUT), jnp.float32),
        acc2, b2.reshape(1, COUT))
    return out
